# baseline (speedup 1.0000x reference)
import functools

import jax
import jax.numpy as jnp
from jax.experimental import pallas as pl
from jax.experimental.pallas import tpu as pltpu
from jax.experimental.pallas import tpu_sc as plsc

N_HEADS = 12
WINDOW = 64
TAU = 2.0
NEG = -1e30


def _rank_kernel(col_full, row_blk, rank_glob, *, T, BR):
    b = pl.program_id(0)
    j0 = pl.program_id(1) * BR
    ck_col = col_full[0, :, :]
    cj_row = row_blk[0, :, :]
    k_col = jax.lax.broadcasted_iota(jnp.int32, (T, 1), 0)
    j_row = j0 + jax.lax.broadcasted_iota(jnp.int32, (1, BR), 1)
    lt = ck_col < cj_row
    eq = (ck_col == cj_row) & (k_col < j_row)
    rank_glob[0, 0, :] = jnp.sum((lt | eq).astype(jnp.int32), axis=0) + b * T


def _sc_scatter(x2d, idx, N, E):
    mesh = plsc.VectorSubcoreMesh(core_axis_name="core",
                                  subcore_axis_name="subcore")
    GW = 128

    @functools.partial(pl.kernel,
                       out_type=jax.ShapeDtypeStruct((N, E), x2d.dtype),
                       mesh=mesh)
    def run(x_hbm, i_hbm, o_hbm):
        def body(x_vmem, i_vmem):
            pltpu.sync_copy(x_vmem, o_hbm.at[i_vmem.at[0]])

        pltpu.emit_pipeline(
            body,
            grid=(N // GW,),
            in_specs=[pl.BlockSpec((GW, E), lambda i: (i, 0)),
                      pl.BlockSpec((1, GW), lambda i: (0, i))],
            out_specs=[],
            core_axis_name=("core", "subcore"),
            dimension_semantics=(pltpu.PARALLEL,),
        )(x_hbm, i_hbm)

    return run(x2d, idx)


def _sc_gather(x2d, idx, N, E):
    mesh = plsc.VectorSubcoreMesh(core_axis_name="core",
                                  subcore_axis_name="subcore")
    GW = 128

    @functools.partial(pl.kernel,
                       out_type=jax.ShapeDtypeStruct((N, E), x2d.dtype),
                       mesh=mesh)
    def run(x_hbm, i_hbm, o_hbm):
        def body(i_vmem, o_vmem):
            pltpu.sync_copy(x_hbm.at[i_vmem.at[0]], o_vmem)

        pltpu.emit_pipeline(
            body,
            grid=(N // GW,),
            in_specs=[pl.BlockSpec((1, GW), lambda i: (0, i))],
            out_specs=[pl.BlockSpec((GW, E), lambda i: (i, 0))],
            core_axis_name=("core", "subcore"),
            dimension_semantics=(pltpu.PARALLEL,),
        )(i_hbm, o_hbm)

    return run(x2d, idx)


def _qkv_kernel(hs_ref, w_ref, b_ref, rank_glob, coord_row,
                q_out, k_out, v_out, cs_out, *, T, BS, H, DH):
    b = pl.program_id(0)
    qs = pl.program_id(1) * BS
    hs = hs_ref[0, :, :]
    for c in range(3 * H):
        chunk = jnp.dot(hs, w_ref[c, :, :],
                        preferred_element_type=jnp.float32) + b_ref[c, :, :]
        if c < H:
            q_out[0, c, :, :] = chunk
        elif c < 2 * H:
            k_out[0, c - H, :, :] = chunk
        else:
            v_out[0, c - 2 * H, :, :] = chunk
    rk = rank_glob[0, :, :]
    tgt = b * T + qs + jax.lax.broadcasted_iota(jnp.int32, (BS, 1), 0)
    sel = rk == tgt
    cs_out[0, :, :] = jnp.sum(jnp.where(sel, coord_row[0, :, :], 0.0),
                              axis=1, keepdims=True)


def _dyn_row(ref, pre, idx):
    base = pl.multiple_of((idx // 8) * 8, 8)
    blk = ref[pre + (pl.ds(base, 8), slice(None))]
    sel = jax.lax.broadcasted_iota(jnp.int32, (8, 1), 0) == (idx - base)
    return jnp.sum(jnp.where(sel, blk, 0.0), axis=0, keepdims=True)


def _attn_kernel(depot_ref, q_ref, k_ref, v_ref, ct_ref, cu_ref, wo_ref,
                 bo_ref, out_ref, *, T, BQ, H, DH, E):
    b = pl.program_id(0)
    qs = pl.program_id(1) * BQ
    d = depot_ref[b]
    scale = 1.0 / (DH ** 0.5)
    inv_tau = 1.0 / TAU
    BK = BQ + WINDOW
    half = WINDOW // 2
    h0 = jnp.clip(qs - half, 0, T - BK)
    h0 = pl.multiple_of(h0, 32)
    ct = ct_ref[0, :, :]
    cu = cu_ref[0, pl.ds(h0, BK), :]
    cu_full = cu_ref[0, :, :]
    cd = _dyn_row(cu_ref, (0,), d)
    t = qs + jax.lax.broadcasted_iota(jnp.int32, (BQ, 1), 0)
    u = h0 + jax.lax.broadcasted_iota(jnp.int32, (1, BK), 1)
    start = jnp.clip(t - half, 0, T - WINDOW)
    mask = (u >= start) & (u < start + WINDOW)
    keep_d = ~((d >= start) & (d < start + WINDOW))
    is_d = t == d
    q_extra = jnp.concatenate(
        [-inv_tau * ct * ct, (2.0 * inv_tau) * ct,
         jnp.full((BQ, 1), -inv_tau, jnp.float32)], axis=1)
    qd_extra = jnp.concatenate(
        [-inv_tau * cd * cd, (2.0 * inv_tau) * cd,
         jnp.full((1, 1), -inv_tau, jnp.float32)], axis=1)
    k_extra = jnp.concatenate(
        [jnp.ones((BK, 1), jnp.float32), cu, cu * cu], axis=1)
    k_extra_full = jnp.concatenate(
        [jnp.ones((T, 1), jnp.float32), cu_full, cu_full * cu_full],
        axis=1)
    kd_extra = jnp.concatenate(
        [jnp.ones((1, 1), jnp.float32), cd, cd * cd], axis=1)
    dq = jnp.clip(d - qs, 0, BQ - 1)

    def dotT(a, bmat):
        return jax.lax.dot_general(a, bmat, (((1,), (1,)), ((), ())),
                                   preferred_element_type=jnp.float32)

    acc = jnp.zeros((BQ, E), jnp.float32)
    for h in range(H):
        q = q_ref[0, h, :, :] * scale
        kh = k_ref[0, h, pl.ds(h0, BK), :]
        vh = v_ref[0, h, pl.ds(h0, BK), :]
        s = dotT(q, kh) + dotT(q_extra, k_extra)
        s = jnp.where(mask, s, NEG)
        kd = _dyn_row(k_ref, (0, h), d)
        sd = dotT(q, kd) + dotT(q_extra, kd_extra)
        sd = jnp.where(keep_d, sd, NEG)
        mx = jnp.maximum(jnp.max(s, axis=1, keepdims=True), sd)
        p = jnp.where(mask, jnp.exp(s - mx), 0.0)
        pd = jnp.where(keep_d, jnp.exp(sd - mx), 0.0)
        dn = jnp.sum(p, axis=1, keepdims=True) + pd
        vd = _dyn_row(v_ref, (0, h), d)
        ctx = (jnp.dot(p, vh, preferred_element_type=jnp.float32)
               + pd * vd) / dn
        qd = _dyn_row(q_ref, (0, h), dq) * scale
        sf = dotT(qd, k_ref[0, h, :, :]) + dotT(qd_extra, k_extra_full)
        mxf = jnp.max(sf, axis=1, keepdims=True)
        pf = jnp.exp(sf - mxf)
        ctx_d = (jnp.dot(pf, v_ref[0, h, :, :],
                         preferred_element_type=jnp.float32)
                 / jnp.sum(pf, axis=1, keepdims=True))
        ctx = jnp.where(is_d, ctx_d, ctx)
        acc = acc + jnp.dot(ctx, wo_ref[h, :, :],
                            preferred_element_type=jnp.float32)
    out_ref[0, :, :] = acc + bo_ref[0, :]


def kernel(h, coord_1d, Wq_w, Wq_b, Wk_w, Wk_b, Wv_w, Wv_b, Wo_w, Wo_b):
    B, T, E = h.shape
    H = N_HEADS
    DH = E // H
    BR = 256
    BS = 256
    BQ = 256
    N = B * T

    coord_row = coord_1d.reshape(B, 1, T)
    coord_col = coord_1d.reshape(B, T, 1)
    w_qkv = jnp.concatenate([Wq_w, Wk_w, Wv_w], axis=1)
    w_heads = w_qkv.reshape(E, 3 * H, DH).transpose(1, 0, 2)
    b_heads = jnp.concatenate([Wq_b, Wk_b, Wv_b]).reshape(3 * H, 1, DH)
    wo_heads = Wo_w.reshape(H, DH, E)
    b_o = Wo_b.reshape(1, E)

    rank_glob = pl.pallas_call(
        functools.partial(_rank_kernel, T=T, BR=BR),
        grid=(B, T // BR),
        in_specs=[
            pl.BlockSpec((1, T, 1), lambda b, j: (b, 0, 0)),
            pl.BlockSpec((1, 1, BR), lambda b, j: (b, 0, j)),
        ],
        out_specs=pl.BlockSpec((1, 1, BR), lambda b, j: (b, 0, j)),
        out_shape=jax.ShapeDtypeStruct((B, 1, T), jnp.int32),
    )(coord_col, coord_row)

    depot = rank_glob[:, 0, 0] - jnp.arange(B, dtype=jnp.int32) * T
    SPLIT = 2
    E2 = E // SPLIT
    N2 = N * SPLIT
    idx = (SPLIT * rank_glob.reshape(N)[:, None]
           + jnp.arange(SPLIT, dtype=jnp.int32)[None, :]).reshape(1, N2)

    h_sorted = _sc_scatter(h.reshape(N2, E2), idx, N2, E2).reshape(B, T, E)

    q_hm, k_hm, v_hm, cs_col = pl.pallas_call(
        functools.partial(_qkv_kernel, T=T, BS=BS, H=H, DH=DH),
        grid=(B, T // BS),
        in_specs=[
            pl.BlockSpec((1, BS, E), lambda b, i: (b, i, 0)),
            pl.BlockSpec((3 * H, E, DH), lambda b, i: (0, 0, 0)),
            pl.BlockSpec((3 * H, 1, DH), lambda b, i: (0, 0, 0)),
            pl.BlockSpec((1, 1, T), lambda b, i: (b, 0, 0)),
            pl.BlockSpec((1, 1, T), lambda b, i: (b, 0, 0)),
        ],
        out_specs=[
            pl.BlockSpec((1, H, BS, DH), lambda b, i: (b, 0, i, 0)),
            pl.BlockSpec((1, H, BS, DH), lambda b, i: (b, 0, i, 0)),
            pl.BlockSpec((1, H, BS, DH), lambda b, i: (b, 0, i, 0)),
            pl.BlockSpec((1, BS, 1), lambda b, i: (b, i, 0)),
        ],
        out_shape=[
            jax.ShapeDtypeStruct((B, H, T, DH), jnp.float32),
            jax.ShapeDtypeStruct((B, H, T, DH), jnp.float32),
            jax.ShapeDtypeStruct((B, H, T, DH), jnp.float32),
            jax.ShapeDtypeStruct((B, T, 1), jnp.float32),
        ],
    )(h_sorted, w_heads, b_heads, rank_glob, coord_row)

    out_sorted = pl.pallas_call(
        functools.partial(_attn_kernel, T=T, BQ=BQ, H=H, DH=DH, E=E),
        grid_spec=pltpu.PrefetchScalarGridSpec(
            num_scalar_prefetch=1,
            grid=(B, T // BQ),
            in_specs=[
                pl.BlockSpec((1, H, BQ, DH), lambda b, i, dref: (b, 0, i, 0)),
                pl.BlockSpec((1, H, T, DH), lambda b, i, dref: (b, 0, 0, 0)),
                pl.BlockSpec((1, H, T, DH), lambda b, i, dref: (b, 0, 0, 0)),
                pl.BlockSpec((1, BQ, 1), lambda b, i, dref: (b, i, 0)),
                pl.BlockSpec((1, T, 1), lambda b, i, dref: (b, 0, 0)),
                pl.BlockSpec((H, DH, E), lambda b, i, dref: (0, 0, 0)),
                pl.BlockSpec((1, E), lambda b, i, dref: (0, 0)),
            ],
            out_specs=pl.BlockSpec((1, BQ, E), lambda b, i, dref: (b, i, 0)),
        ),
        out_shape=jax.ShapeDtypeStruct((B, T, E), jnp.float32),
        compiler_params=pltpu.CompilerParams(
            vmem_limit_bytes=64 * 1024 * 1024),
    )(depot, q_hm, k_hm, v_hm, cs_col, cs_col, wo_heads, b_o)

    out = _sc_gather(out_sorted.reshape(N2, E2), idx, N2, E2).reshape(B, T, E)
    return out


# speedup vs baseline: 1.2754x; 1.2754x over previous
import functools

import jax
import jax.numpy as jnp
from jax.experimental import pallas as pl
from jax.experimental.pallas import tpu as pltpu
from jax.experimental.pallas import tpu_sc as plsc

N_HEADS = 12
WINDOW = 64
TAU = 2.0
NEG = -1e30


def _rank_kernel(col_full, row_blk, rank_glob, *, T, BR):
    b = pl.program_id(0)
    j0 = pl.program_id(1) * BR
    ck_col = col_full[0, :, :]
    cj_row = row_blk[0, :, :]
    k_col = jax.lax.broadcasted_iota(jnp.int32, (T, 1), 0)
    j_row = j0 + jax.lax.broadcasted_iota(jnp.int32, (1, BR), 1)
    lt = ck_col < cj_row
    eq = (ck_col == cj_row) & (k_col < j_row)
    rank_glob[0, 0, :] = jnp.sum((lt | eq).astype(jnp.int32), axis=0) + b * T


def _sc_scatter(x2d, idx, N, E):
    mesh = plsc.VectorSubcoreMesh(core_axis_name="core",
                                  subcore_axis_name="subcore")
    GW = 128

    @functools.partial(pl.kernel,
                       out_type=jax.ShapeDtypeStruct((N, E), x2d.dtype),
                       mesh=mesh)
    def run(x_hbm, i_hbm, o_hbm):
        def body(x_vmem, i_vmem):
            pltpu.sync_copy(x_vmem, o_hbm.at[i_vmem.at[0]])

        pltpu.emit_pipeline(
            body,
            grid=(N // GW,),
            in_specs=[pl.BlockSpec((GW, E), lambda i: (i, 0)),
                      pl.BlockSpec((1, GW), lambda i: (0, i))],
            out_specs=[],
            core_axis_name=("core", "subcore"),
            dimension_semantics=(pltpu.PARALLEL,),
        )(x_hbm, i_hbm)

    return run(x2d, idx)


def _sc_gather(x2d, idx, N, E):
    mesh = plsc.VectorSubcoreMesh(core_axis_name="core",
                                  subcore_axis_name="subcore")
    GW = 128

    @functools.partial(pl.kernel,
                       out_type=jax.ShapeDtypeStruct((N, E), x2d.dtype),
                       mesh=mesh)
    def run(x_hbm, i_hbm, o_hbm):
        def body(i_vmem, o_vmem):
            pltpu.sync_copy(x_hbm.at[i_vmem.at[0]], o_vmem)

        pltpu.emit_pipeline(
            body,
            grid=(N // GW,),
            in_specs=[pl.BlockSpec((1, GW), lambda i: (0, i))],
            out_specs=[pl.BlockSpec((GW, E), lambda i: (i, 0))],
            core_axis_name=("core", "subcore"),
            dimension_semantics=(pltpu.PARALLEL,),
        )(i_hbm, o_hbm)

    return run(x2d, idx)


def _qkv_kernel(hs_ref, w_ref, b_ref, rank_glob, coord_row, qkv_out, cs_out,
                *, T, BS):
    b = pl.program_id(0)
    qs = pl.program_id(1) * BS
    qkv_out[0, :, :] = jnp.dot(hs_ref[0, :, :], w_ref[:, :],
                               preferred_element_type=jnp.float32) + b_ref[0, :]
    rk = rank_glob[0, :, :]
    tgt = b * T + qs + jax.lax.broadcasted_iota(jnp.int32, (BS, 1), 0)
    sel = rk == tgt
    cs_out[0, :, :] = jnp.sum(jnp.where(sel, coord_row[0, :, :], 0.0),
                              axis=1, keepdims=True)


def _dyn_row(ref, pre, idx):
    base = pl.multiple_of((idx // 8) * 8, 8)
    blk = ref[pre + (pl.ds(base, 8), slice(None))]
    sel = jax.lax.broadcasted_iota(jnp.int32, (8, 1), 0) == (idx - base)
    return jnp.sum(jnp.where(sel, blk, 0.0), axis=0, keepdims=True)


def _dotT(a, bmat):
    return jax.lax.dot_general(a, bmat, (((1,), (1,)), ((), ())),
                               preferred_element_type=jnp.float32)


def _depot_kernel(depot_ref, qkv_ref, cu_ref, wo_ref, ctxd_out,
                  *, T, H, DH, E):
    b = pl.program_id(0)
    d = depot_ref[b]
    scale = 1.0 / (DH ** 0.5)
    inv_tau = 1.0 / TAU
    cu_full = cu_ref[0, :, :]
    cd = _dyn_row(cu_ref, (0,), d)
    qd_extra = jnp.concatenate(
        [-inv_tau * cd * cd, (2.0 * inv_tau) * cd,
         jnp.full((1, 1), -inv_tau, jnp.float32)], axis=1)
    k_extra_full = jnp.concatenate(
        [jnp.ones((T, 1), jnp.float32), cu_full, cu_full * cu_full],
        axis=1)
    qd_all = _dyn_row(qkv_ref, (0,), d)
    acc = jnp.zeros((1, E), jnp.float32)
    for h in range(H):
        lo, hi = h * DH, (h + 1) * DH
        qd = qd_all[:, lo:hi] * scale
        kh = qkv_ref[0, :, E + lo:E + hi]
        vh = qkv_ref[0, :, 2 * E + lo:2 * E + hi]
        sf = _dotT(qd, kh) + _dotT(qd_extra, k_extra_full)
        mxf = jnp.max(sf, axis=1, keepdims=True)
        pf = jnp.exp(sf - mxf)
        ctx_d = (jnp.dot(pf, vh, preferred_element_type=jnp.float32)
                 / jnp.sum(pf, axis=1, keepdims=True))
        acc = acc + jnp.dot(ctx_d, wo_ref[lo:hi, :],
                            preferred_element_type=jnp.float32)
    ctxd_out[0, :, :] = acc


def _attn_kernel(depot_ref, q_ref, k_ref, v_ref, ct_ref, cu_ref, wo_ref,
                 bo_ref, ctxd_ref, out_ref, *, T, BQ, H, DH, E):
    b = pl.program_id(0)
    qs = pl.program_id(1) * BQ
    d = depot_ref[b]
    scale = 1.0 / (DH ** 0.5)
    inv_tau = 1.0 / TAU
    BK = BQ + WINDOW
    half = WINDOW // 2
    h0 = jnp.clip(qs - half, 0, T - BK)
    h0 = pl.multiple_of(h0, 32)
    ct = ct_ref[0, :, :]
    cu = cu_ref[0, pl.ds(h0, BK), :]
    cd = _dyn_row(cu_ref, (0,), d)
    t = qs + jax.lax.broadcasted_iota(jnp.int32, (BQ, 1), 0)
    u = h0 + jax.lax.broadcasted_iota(jnp.int32, (1, BK), 1)
    start = jnp.clip(t - half, 0, T - WINDOW)
    mask = (u >= start) & (u < start + WINDOW)
    keep_d = ~((d >= start) & (d < start + WINDOW))
    is_d = t == d
    q_extra = jnp.concatenate(
        [-inv_tau * ct * ct, (2.0 * inv_tau) * ct,
         jnp.full((BQ, 1), -inv_tau, jnp.float32)], axis=1)
    k_extra = jnp.concatenate(
        [jnp.ones((BK, 1), jnp.float32), cu, cu * cu], axis=1)
    kd_extra = jnp.concatenate(
        [jnp.ones((1, 1), jnp.float32), cd, cd * cd], axis=1)
    kd_all = _dyn_row(k_ref, (0,), d)
    vd_all = _dyn_row(v_ref, (0,), d)
    acc = jnp.zeros((BQ, E), jnp.float32)
    for h in range(H):
        lo, hi = h * DH, (h + 1) * DH
        q = q_ref[0, :, lo:hi] * scale
        kh = k_ref[0, pl.ds(h0, BK), lo:hi]
        vh = v_ref[0, pl.ds(h0, BK), lo:hi]
        s = _dotT(q, kh) + _dotT(q_extra, k_extra)
        s = jnp.where(mask, s, NEG)
        sd = _dotT(q, kd_all[:, lo:hi]) + _dotT(q_extra, kd_extra)
        sd = jnp.where(keep_d, sd, NEG)
        mx = jnp.maximum(jnp.max(s, axis=1, keepdims=True), sd)
        p = jnp.where(mask, jnp.exp(s - mx), 0.0)
        pd = jnp.where(keep_d, jnp.exp(sd - mx), 0.0)
        dn = jnp.sum(p, axis=1, keepdims=True) + pd
        ctx = (jnp.dot(p, vh, preferred_element_type=jnp.float32)
               + pd * vd_all[:, lo:hi]) / dn
        acc = acc + jnp.dot(ctx, wo_ref[lo:hi, :],
                            preferred_element_type=jnp.float32)
    acc = jnp.where(is_d, ctxd_ref[0, :, :], acc)
    out_ref[0, :, :] = acc + bo_ref[0, :]


def kernel(h, coord_1d, Wq_w, Wq_b, Wk_w, Wk_b, Wv_w, Wv_b, Wo_w, Wo_b):
    B, T, E = h.shape
    H = N_HEADS
    DH = E // H
    BR = 256
    BS = 256
    BQ = 256
    N = B * T

    coord_row = coord_1d.reshape(B, 1, T)
    coord_col = coord_1d.reshape(B, T, 1)
    w_qkv = jnp.concatenate([Wq_w, Wk_w, Wv_w], axis=1)
    b_qkv = jnp.concatenate([Wq_b, Wk_b, Wv_b]).reshape(1, 3 * E)
    b_o = Wo_b.reshape(1, E)

    rank_glob = pl.pallas_call(
        functools.partial(_rank_kernel, T=T, BR=BR),
        grid=(B, T // BR),
        in_specs=[
            pl.BlockSpec((1, T, 1), lambda b, j: (b, 0, 0)),
            pl.BlockSpec((1, 1, BR), lambda b, j: (b, 0, j)),
        ],
        out_specs=pl.BlockSpec((1, 1, BR), lambda b, j: (b, 0, j)),
        out_shape=jax.ShapeDtypeStruct((B, 1, T), jnp.int32),
    )(coord_col, coord_row)

    depot = rank_glob[:, 0, 0] - jnp.arange(B, dtype=jnp.int32) * T
    SPLIT = 2
    E2 = E // SPLIT
    N2 = N * SPLIT
    idx = (SPLIT * rank_glob.reshape(N)[:, None]
           + jnp.arange(SPLIT, dtype=jnp.int32)[None, :]).reshape(1, N2)

    h_sorted = _sc_scatter(h.reshape(N2, E2), idx, N2, E2).reshape(B, T, E)

    qkv, cs_col = pl.pallas_call(
        functools.partial(_qkv_kernel, T=T, BS=BS),
        grid=(B, T // BS),
        in_specs=[
            pl.BlockSpec((1, BS, E), lambda b, i: (b, i, 0)),
            pl.BlockSpec((E, 3 * E), lambda b, i: (0, 0)),
            pl.BlockSpec((1, 3 * E), lambda b, i: (0, 0)),
            pl.BlockSpec((1, 1, T), lambda b, i: (b, 0, 0)),
            pl.BlockSpec((1, 1, T), lambda b, i: (b, 0, 0)),
        ],
        out_specs=[
            pl.BlockSpec((1, BS, 3 * E), lambda b, i: (b, i, 0)),
            pl.BlockSpec((1, BS, 1), lambda b, i: (b, i, 0)),
        ],
        out_shape=[
            jax.ShapeDtypeStruct((B, T, 3 * E), jnp.float32),
            jax.ShapeDtypeStruct((B, T, 1), jnp.float32),
        ],
    )(h_sorted, w_qkv, b_qkv, rank_glob, coord_row)

    ctxd = pl.pallas_call(
        functools.partial(_depot_kernel, T=T, H=H, DH=DH, E=E),
        grid_spec=pltpu.PrefetchScalarGridSpec(
            num_scalar_prefetch=1,
            grid=(B,),
            in_specs=[
                pl.BlockSpec((1, T, 3 * E), lambda b, dref: (b, 0, 0)),
                pl.BlockSpec((1, T, 1), lambda b, dref: (b, 0, 0)),
                pl.BlockSpec((E, E), lambda b, dref: (0, 0)),
            ],
            out_specs=pl.BlockSpec((1, 1, E), lambda b, dref: (b, 0, 0)),
        ),
        out_shape=jax.ShapeDtypeStruct((B, 1, E), jnp.float32),
        compiler_params=pltpu.CompilerParams(
            vmem_limit_bytes=64 * 1024 * 1024),
    )(depot, qkv, cs_col, Wo_w)

    out_sorted = pl.pallas_call(
        functools.partial(_attn_kernel, T=T, BQ=BQ, H=H, DH=DH, E=E),
        grid_spec=pltpu.PrefetchScalarGridSpec(
            num_scalar_prefetch=1,
            grid=(B, T // BQ),
            in_specs=[
                pl.BlockSpec((1, BQ, E), lambda b, i, dref: (b, i, 0)),
                pl.BlockSpec((1, T, E), lambda b, i, dref: (b, 0, 1)),
                pl.BlockSpec((1, T, E), lambda b, i, dref: (b, 0, 2)),
                pl.BlockSpec((1, BQ, 1), lambda b, i, dref: (b, i, 0)),
                pl.BlockSpec((1, T, 1), lambda b, i, dref: (b, 0, 0)),
                pl.BlockSpec((E, E), lambda b, i, dref: (0, 0)),
                pl.BlockSpec((1, E), lambda b, i, dref: (0, 0)),
                pl.BlockSpec((1, 1, E), lambda b, i, dref: (b, 0, 0)),
            ],
            out_specs=pl.BlockSpec((1, BQ, E), lambda b, i, dref: (b, i, 0)),
        ),
        out_shape=jax.ShapeDtypeStruct((B, T, E), jnp.float32),
        compiler_params=pltpu.CompilerParams(
            vmem_limit_bytes=64 * 1024 * 1024),
    )(depot, qkv, qkv, qkv, cs_col, cs_col, Wo_w, b_o, ctxd)

    out = _sc_gather(out_sorted.reshape(N2, E2), idx, N2, E2).reshape(B, T, E)
    return out
